# baseline (speedup 1.0000x reference)
import functools
import math

import jax
import jax.numpy as jnp
from jax import lax
from jax.experimental import pallas as pl
from jax.experimental.pallas import tpu as pltpu
from jax.experimental.pallas import tpu_sc as plsc

G = 64
GC = 32
N_PTS = 400000
C_IN = 32
C_OUT = 64
NVOX = GC * GC * GC
EPS = 1e-5


def _matmul_kern(x_ref, w_ref, o_ref):
    o_ref[...] = jnp.dot(x_ref[...], w_ref[...],
                         preferred_element_type=jnp.float32)


def _conv_in_matmul(Mb, Wf):
    bm = 4096
    return pl.pallas_call(
        _matmul_kern,
        grid=(NVOX // bm,),
        in_specs=[pl.BlockSpec((bm, 256), lambda i: (i, 0)),
                  pl.BlockSpec((256, C_OUT), lambda i: (0, 0))],
        out_specs=pl.BlockSpec((bm, C_OUT), lambda i: (i, 0)),
        out_shape=jax.ShapeDtypeStruct((NVOX, C_OUT), jnp.float32),
    )(Mb, Wf)


def _bn_relu_kern(x_ref, g_ref, b_ref, of_ref, ob_ref):
    x = x_ref[...]
    n = x.shape[0]
    m = jnp.sum(x, axis=0, keepdims=True) / n
    v = jnp.sum(x * x, axis=0, keepdims=True) / n - m * m
    s = g_ref[...] * lax.rsqrt(v + EPS)
    y = jnp.maximum(x * s + (b_ref[...] - m * s), 0.0)
    of_ref[...] = y
    ob_ref[...] = y.astype(jnp.bfloat16)


def _bn_relu(x, g, b):
    return pl.pallas_call(
        _bn_relu_kern,
        out_shape=(jax.ShapeDtypeStruct((NVOX, C_OUT), jnp.float32),
                   jax.ShapeDtypeStruct((NVOX, C_OUT), jnp.bfloat16)),
    )(x, g.reshape(1, C_OUT), b.reshape(1, C_OUT))


def _bn_add_relu_kern(x_ref, r_ref, g_ref, b_ref, ob_ref):
    x = x_ref[...]
    n = x.shape[0]
    m = jnp.sum(x, axis=0, keepdims=True) / n
    v = jnp.sum(x * x, axis=0, keepdims=True) / n - m * m
    s = g_ref[...] * lax.rsqrt(v + EPS)
    y = jnp.maximum(x * s + (b_ref[...] - m * s) + r_ref[...], 0.0)
    ob_ref[...] = y.astype(jnp.bfloat16)


def _bn_add_relu(x, r, g, b):
    return pl.pallas_call(
        _bn_add_relu_kern,
        out_shape=jax.ShapeDtypeStruct((NVOX, C_OUT), jnp.bfloat16),
    )(x, r, g.reshape(1, C_OUT), b.reshape(1, C_OUT))


def _conv3_kern(a0_ref, a1_ref, a2_ref, w_ref, o_ref):
    acc = jnp.dot(a0_ref[...].reshape(GC * GC, 576), w_ref[0],
                  preferred_element_type=jnp.float32)
    acc += jnp.dot(a1_ref[...].reshape(GC * GC, 576), w_ref[1],
                   preferred_element_type=jnp.float32)
    acc += jnp.dot(a2_ref[...].reshape(GC * GC, 576), w_ref[2],
                   preferred_element_type=jnp.float32)
    o_ref[...] = acc


def _conv3(a_bf, W):
    ap = jnp.pad(a_bf, ((1, 1), (1, 1), (1, 1), (0, 0)))
    ayz = jnp.concatenate(
        [ap[:, dy:dy + 32, dz:dz + 32, :] for dy in range(3)
         for dz in range(3)], axis=-1)
    Wc = W.reshape(3, 576, C_OUT).astype(jnp.bfloat16)
    pspec = lambda k: pl.BlockSpec((1, GC, GC, 576),
                                   lambda x: (x + k, 0, 0, 0))
    return pl.pallas_call(
        _conv3_kern,
        grid=(GC,),
        in_specs=[pspec(0), pspec(1), pspec(2),
                  pl.BlockSpec((3, 576, C_OUT), lambda x: (0, 0, 0))],
        out_specs=pl.BlockSpec((GC * GC, C_OUT), lambda x: (x, 0)),
        out_shape=jax.ShapeDtypeStruct((NVOX, C_OUT), jnp.float32),
    )(ayz, ayz, ayz, Wc)


def _point_stats_kern(x_ref, s1_ref, s2_ref):
    @pl.when(pl.program_id(0) == 0)
    def _():
        s1_ref[...] = jnp.zeros_like(s1_ref)
        s2_ref[...] = jnp.zeros_like(s2_ref)
    x = x_ref[...]
    s1_ref[...] += jnp.sum(x, axis=0, keepdims=True)
    s2_ref[...] += lax.dot_general(x, x, (((0,), (0,)), ((), ())),
                                   preferred_element_type=jnp.float32)


def _point_stats(x_F):
    bm = 16000
    return pl.pallas_call(
        _point_stats_kern,
        grid=(N_PTS // bm,),
        in_specs=[pl.BlockSpec((bm, C_IN), lambda i: (i, 0))],
        out_specs=(pl.BlockSpec((1, C_IN), lambda i: (0, 0)),
                   pl.BlockSpec((C_IN, C_IN), lambda i: (0, 0))),
        out_shape=(jax.ShapeDtypeStruct((1, C_IN), jnp.float32),
                   jax.ShapeDtypeStruct((C_IN, C_IN), jnp.float32)),
    )(x_F)


def _point_mm_kern(x_ref, w_ref, b_ref, o_ref):
    y = jnp.dot(x_ref[...], w_ref[...], preferred_element_type=jnp.float32)
    o_ref[...] = jnp.maximum(y + b_ref[...], 0.0)


def _point_layer_out(x_F, W2, b2):
    bm = 16000
    return pl.pallas_call(
        _point_mm_kern,
        grid=(N_PTS // bm,),
        in_specs=[pl.BlockSpec((bm, C_IN), lambda i: (i, 0)),
                  pl.BlockSpec((C_IN, C_OUT), lambda i: (0, 0)),
                  pl.BlockSpec((1, C_OUT), lambda i: (0, 0))],
        out_specs=pl.BlockSpec((bm, C_OUT), lambda i: (i, 0)),
        out_shape=jax.ShapeDtypeStruct((N_PTS, C_OUT), jnp.float32),
    )(x_F, W2, b2)


def _voxelize_fine_means(x_F, x_C):
    idx = jnp.clip(jnp.floor(x_C).astype(jnp.int32), 0, G - 1)
    flat = (idx[:, 0] * G + idx[:, 1]) * G + idx[:, 2]
    sums = jnp.zeros((G * G * G, C_IN), x_F.dtype).at[flat].add(x_F)
    cnt = jnp.zeros((G * G * G,), x_F.dtype).at[flat].add(1.0)
    grid = sums / jnp.maximum(cnt, 1.0)[:, None]
    t = grid.reshape(GC, 2, GC, 2, GC, 2, C_IN)
    t = t.transpose(0, 2, 4, 1, 3, 5, 6).reshape(NVOX, 8 * C_IN)
    return t.astype(jnp.bfloat16)


def _trilinear_add(v_bf, x_C, po):
    gflat = v_bf.reshape(GC, GC, GC, C_OUT).astype(jnp.float32)
    p = x_C / 2.0
    p0f = jnp.floor(p)
    frac = p - p0f
    p0 = p0f.astype(jnp.int32)
    out = po
    for dx in (0, 1):
        wx = frac[:, 0] if dx else (1.0 - frac[:, 0])
        ix = jnp.clip(p0[:, 0] + dx, 0, GC - 1)
        for dy in (0, 1):
            wy = frac[:, 1] if dy else (1.0 - frac[:, 1])
            iy = jnp.clip(p0[:, 1] + dy, 0, GC - 1)
            for dz in (0, 1):
                wz = frac[:, 2] if dz else (1.0 - frac[:, 2])
                iz = jnp.clip(p0[:, 2] + dz, 0, GC - 1)
                out = out + gflat[ix, iy, iz] * (wx * wy * wz)[:, None]
    return out


def _voxelize_coarse(pf, x_C):
    idx = jnp.clip(jnp.floor(x_C / 2.0).astype(jnp.int32), 0, GC - 1)
    flat = (idx[:, 0] * GC + idx[:, 1]) * GC + idx[:, 2]
    sums = jnp.zeros((NVOX, C_OUT), pf.dtype).at[flat].add(pf)
    cnt = jnp.zeros((NVOX,), pf.dtype).at[flat].add(1.0)
    grid = sums / jnp.maximum(cnt, 1.0)[:, None]
    return grid.reshape(1, GC, GC, GC, C_OUT)


def kernel(x_F, x_C, W_conv_in, bn1_g, bn1_b, W_res1, bn2_g, bn2_b,
           W_res2, bn3_g, bn3_b, W_point, b_point, bnp_g, bnp_b):
    Mb = _voxelize_fine_means(x_F, x_C)

    Wf = W_conv_in.reshape(8 * C_IN, C_OUT).astype(jnp.bfloat16)
    c1 = _conv_in_matmul(Mb, Wf)
    v0, v0b = _bn_relu(c1, bn1_g, bn1_b)

    h1 = _conv3(v0b.reshape(GC, GC, GC, C_OUT), W_res1)
    _, h1b = _bn_relu(h1, bn2_g, bn2_b)
    h2 = _conv3(h1b.reshape(GC, GC, GC, C_OUT), W_res2)
    vb = _bn_add_relu(h2, v0, bn3_g, bn3_b)

    s1, s2 = _point_stats(x_F)
    mx = s1[0] / N_PTS
    M2 = s2 / N_PTS
    U = mx @ W_point
    q = jnp.sum(W_point * (M2 @ W_point), axis=0)
    var = q - U * U
    rs = bnp_g * lax.rsqrt(var + EPS)
    W2 = W_point * rs[None, :]
    b2 = b_point * rs + bnp_b - (U + b_point) * rs
    po = _point_layer_out(x_F, W2, b2.reshape(1, C_OUT))

    pf = _trilinear_add(vb, x_C, po)

    v_new = _voxelize_coarse(pf, x_C)
    return (v_new, pf)
